# baseline (speedup 1.0000x reference)
import functools

import jax
import jax.numpy as jnp
from jax import lax
from jax.experimental import pallas as pl
from jax.experimental.pallas import tpu as pltpu
from jax.experimental.pallas import tpu_sc as plsc

N = 10000
E = 160000
G = 64
N_PAD = 10240
E_PAD = 163840
CHUNK = 128
NW = 32
NSUB = 16
EP_W = E_PAD // NW
NCHUNK = EP_W // CHUNK
ROWS_W = N_PAD // NSUB
BN = 2048
NBLK = N_PAD // BN


def _sc_mesh():
    return plsc.VectorSubcoreMesh(core_axis_name="c", subcore_axis_name="s",
                                  num_cores=2, num_subcores=NSUB)


K_PIPE = 8


def _sc_degrees(edges3, ones_pat, zeros16):
    @functools.partial(
        pl.kernel,
        out_type=jax.ShapeDtypeStruct((2, N_PAD, 16), jnp.float32),
        mesh=_sc_mesh(),
        scratch_types=[
            pltpu.VMEM((K_PIPE, 2, CHUNK), jnp.int32),
            pltpu.VMEM((CHUNK, 16), jnp.float32),
            pltpu.VMEM((CHUNK, 16), jnp.float32),
            [pltpu.SemaphoreType.DMA] * K_PIPE,
            pltpu.SemaphoreType.DMA,
            pltpu.VMEM_SHARED((N_PAD, 16), jnp.float32),
        ],
    )
    def k(ed_ref, ones_ref, zero_ref, out_ref, idx, e_src, e_dst, sem_i,
          sem_sc, acc):
        cid = lax.axis_index("c")
        sid = lax.axis_index("s")
        cbase = (cid * NSUB + sid) * NCHUNK
        r0 = sid * ROWS_W
        pltpu.sync_copy(ones_ref.at[0], e_src)
        pltpu.sync_copy(ones_ref.at[1], e_dst)
        pltpu.sync_copy(zero_ref.at[pl.ds(r0, ROWS_W)], acc.at[pl.ds(r0, ROWS_W)])
        plsc.subcore_barrier()

        def body(j, carry):
            c0 = cbase + j * K_PIPE
            loads = [
                pltpu.async_copy(ed_ref.at[c0 + s], idx.at[s], sem_i[s])
                for s in range(K_PIPE)
            ]
            scats = []
            for s in range(K_PIPE):
                loads[s].wait()
                scats.append(pltpu.async_copy(
                    e_src, acc.at[idx.at[s, 0]], sem_sc, add=True))
                scats.append(pltpu.async_copy(
                    e_dst, acc.at[idx.at[s, 1]], sem_sc, add=True))
            for d in scats:
                d.wait()
            return carry

        lax.fori_loop(0, NCHUNK // K_PIPE, body, 0)
        plsc.subcore_barrier()
        pltpu.sync_copy(acc.at[pl.ds(r0, ROWS_W)], out_ref.at[cid, pl.ds(r0, ROWS_W)])

    return k(edges3, ones_pat, zeros16)


def _sc_scatter(p_hbm, edges3, zeros, dout):
    @functools.partial(
        pl.kernel,
        out_type=jax.ShapeDtypeStruct((2, N_PAD, dout), jnp.float32),
        mesh=_sc_mesh(),
        compiler_params=pltpu.CompilerParams(use_tc_tiling_on_sc=False),
        scratch_types=[
            pltpu.VMEM((K_PIPE, 2, CHUNK), jnp.int32),
            [pltpu.VMEM((CHUNK, dout), jnp.float32)] * K_PIPE,
            [pltpu.SemaphoreType.DMA] * K_PIPE,
            [pltpu.SemaphoreType.DMA] * K_PIPE,
            pltpu.SemaphoreType.DMA,
            pltpu.VMEM_SHARED((N_PAD, dout), jnp.float32),
        ],
    )
    def k(p_ref, ed_ref, zero_ref, out_ref, idx, rows, sem_i, sem_g, sem_sc,
          acc):
        cid = lax.axis_index("c")
        sid = lax.axis_index("s")
        cbase = (cid * NSUB + sid) * NCHUNK
        r0 = sid * ROWS_W
        pltpu.sync_copy(zero_ref.at[pl.ds(r0, ROWS_W)], acc.at[pl.ds(r0, ROWS_W)])
        plsc.subcore_barrier()

        def body(j, carry):
            c0 = cbase + j * K_PIPE
            loads = [
                pltpu.async_copy(ed_ref.at[c0 + s], idx.at[s], sem_i[s])
                for s in range(K_PIPE)
            ]
            gathers = []
            for s in range(K_PIPE):
                loads[s].wait()
                gathers.append(pltpu.async_copy(
                    p_ref.at[idx.at[s, 0]], rows[s], sem_g[s]))
            scats = []
            for s in range(K_PIPE):
                gathers[s].wait()
                scats.append(pltpu.async_copy(
                    rows[s], acc.at[idx.at[s, 1]], sem_sc, add=True))
            for d in scats:
                d.wait()
            return carry

        lax.fori_loop(0, NCHUNK // K_PIPE, body, 0)
        plsc.subcore_barrier()
        pltpu.sync_copy(acc.at[pl.ds(r0, ROWS_W)], out_ref.at[cid, pl.ds(r0, ROWS_W)])

    return k(p_hbm, edges3, zeros)


def _tc_pre(x_pad, W1_0, W2_0, b_0):
    def body(x_ref, w1_ref, w2_ref, bias_ref, w1x_ref, q_ref):
        xb = x_ref[...]
        w1x_ref[...] = jnp.dot(xb, w1_ref[...],
                               preferred_element_type=jnp.float32)
        q_ref[...] = jnp.dot(xb, w2_ref[...],
                             preferred_element_type=jnp.float32) + bias_ref[...]

    return pl.pallas_call(
        body,
        grid=(NBLK,),
        in_specs=[
            pl.BlockSpec((BN, 256), lambda i: (i, 0)),
            pl.BlockSpec((256, 64), lambda i: (0, 0)),
            pl.BlockSpec((256, 64), lambda i: (0, 0)),
            pl.BlockSpec((1, 64), lambda i: (0, 0)),
        ],
        out_specs=[
            pl.BlockSpec((BN, 64), lambda i: (i, 0)),
            pl.BlockSpec((BN, 64), lambda i: (i, 0)),
        ],
        out_shape=[
            jax.ShapeDtypeStruct((N_PAD, 64), jnp.float32),
            jax.ShapeDtypeStruct((N_PAD, 64), jnp.float32),
        ],
    )(x_pad, W1_0, W2_0, b_0)


def _tc_prep(d0, d1, w1x):
    def body(d0_ref, d1_ref, w1x_ref, a_ref, b_ref, p_ref):
        deg = d0_ref[...] + d1_ref[...]
        a = lax.rsqrt(jnp.maximum(deg[:, 0:1], 1.0))
        b = lax.rsqrt(jnp.maximum(deg[:, 1:2], 1.0))
        a_ref[...] = a
        b_ref[...] = b
        p_ref[...] = w1x_ref[...] * a

    return pl.pallas_call(
        body,
        grid=(NBLK,),
        in_specs=[
            pl.BlockSpec((BN, 16), lambda i: (i, 0)),
            pl.BlockSpec((BN, 16), lambda i: (i, 0)),
            pl.BlockSpec((BN, 64), lambda i: (i, 0)),
        ],
        out_specs=[
            pl.BlockSpec((BN, 1), lambda i: (i, 0)),
            pl.BlockSpec((BN, 1), lambda i: (i, 0)),
            pl.BlockSpec((BN, 64), lambda i: (i, 0)),
        ],
        out_shape=[
            jax.ShapeDtypeStruct((N_PAD, 1), jnp.float32),
            jax.ShapeDtypeStruct((N_PAD, 1), jnp.float32),
            jax.ShapeDtypeStruct((N_PAD, 64), jnp.float32),
        ],
    )(d0, d1, w1x)


def _tc_post(acc0, acc1, bvec, q, avec, W1n, W2n, biasn, dout, dnext):
    def body(a0_ref, a1_ref, b_ref, q_ref, a_ref, w1n_ref, w2n_ref, bias_ref,
             p_ref, qn_ref):
        i = pl.program_id(0)
        rows = lax.broadcasted_iota(jnp.int32, (BN, 1), 0) + i * BN
        msk = (rows < N).astype(jnp.float32)
        agg = (a0_ref[...] + a1_ref[...]) * b_ref[...]
        h = jnp.maximum(agg + q_ref[...], 0.0) * msk
        p_ref[...] = jnp.dot(h, w1n_ref[...],
                             preferred_element_type=jnp.float32) * a_ref[...]
        qn_ref[...] = jnp.dot(h, w2n_ref[...],
                              preferred_element_type=jnp.float32) + bias_ref[...]

    return pl.pallas_call(
        body,
        grid=(NBLK,),
        in_specs=[
            pl.BlockSpec((BN, dout), lambda i: (i, 0)),
            pl.BlockSpec((BN, dout), lambda i: (i, 0)),
            pl.BlockSpec((BN, 1), lambda i: (i, 0)),
            pl.BlockSpec((BN, dout), lambda i: (i, 0)),
            pl.BlockSpec((BN, 1), lambda i: (i, 0)),
            pl.BlockSpec((dout, dnext), lambda i: (0, 0)),
            pl.BlockSpec((dout, dnext), lambda i: (0, 0)),
            pl.BlockSpec((1, dnext), lambda i: (0, 0)),
        ],
        out_specs=[
            pl.BlockSpec((BN, dnext), lambda i: (i, 0)),
            pl.BlockSpec((BN, dnext), lambda i: (i, 0)),
        ],
        out_shape=[
            jax.ShapeDtypeStruct((N_PAD, dnext), jnp.float32),
            jax.ShapeDtypeStruct((N_PAD, dnext), jnp.float32),
        ],
    )(acc0, acc1, bvec, q, avec, W1n, W2n, biasn)


def _tc_final(acc0, acc1, bvec, q3, seg2, Wo, bo):
    def body(a0_ref, a1_ref, b_ref, q_ref, seg_ref, wo_ref,
             bo_ref, out_ref, sums, counts):
        i = pl.program_id(0)

        @pl.when(i == 0)
        def _():
            sums[...] = jnp.zeros_like(sums)
            counts[...] = jnp.zeros_like(counts)

        rows = lax.broadcasted_iota(jnp.int32, (BN, 1), 0) + i * BN
        msk = (rows < N).astype(jnp.float32)
        agg = (a0_ref[...] + a1_ref[...]) * b_ref[...]
        h = jnp.maximum(agg + q_ref[...], 0.0) * msk
        gid = lax.broadcasted_iota(jnp.int32, (BN, G), 1)
        onehot = (seg_ref[...] == gid).astype(jnp.float32)
        sums[...] += lax.dot_general(onehot, h, (((0,), (0,)), ((), ())),
                                     preferred_element_type=jnp.float32)
        counts[...] += lax.dot_general(onehot, jnp.ones((BN, 16), jnp.float32),
                                       (((0,), (0,)), ((), ())),
                                       preferred_element_type=jnp.float32)

        @pl.when(i == NBLK - 1)
        def _():
            pooled = sums[...] / jnp.maximum(counts[...], 1.0)
            out_ref[...] = jnp.dot(pooled, wo_ref[...],
                                   preferred_element_type=jnp.float32) + bo_ref[...]

    return pl.pallas_call(
        body,
        grid=(NBLK,),
        in_specs=[
            pl.BlockSpec((BN, 16), lambda i: (i, 0)),
            pl.BlockSpec((BN, 16), lambda i: (i, 0)),
            pl.BlockSpec((BN, 1), lambda i: (i, 0)),
            pl.BlockSpec((BN, 16), lambda i: (i, 0)),
            pl.BlockSpec((BN, 1), lambda i: (i, 0)),
            pl.BlockSpec((16, 1), lambda i: (0, 0)),
            pl.BlockSpec((1, 1), lambda i: (0, 0)),
        ],
        out_specs=pl.BlockSpec((G, 1), lambda i: (0, 0)),
        out_shape=jax.ShapeDtypeStruct((G, 1), jnp.float32),
        scratch_shapes=[
            pltpu.VMEM((G, 16), jnp.float32),
            pltpu.VMEM((G, 16), jnp.float32),
        ],
    )(acc0, acc1, bvec, q3, seg2, Wo, bo)


def kernel(x, edge_index, segment_ids, W1_0, W2_0, b_0, W1_1, W2_1, b_1,
           W1_2, W2_2, b_2, W1_3, W2_3, b_3, Wo, bo):
    f32 = jnp.float32
    i32 = jnp.int32
    pad_e = jnp.full((2, E_PAD - E), N, i32)
    edges3 = jnp.concatenate([edge_index.astype(i32), pad_e], axis=1) \
        .reshape(2, E_PAD // CHUNK, CHUNK).transpose(1, 0, 2)
    x_pad = jnp.pad(x, ((0, N_PAD - N), (0, 0)))
    seg2 = jnp.concatenate(
        [segment_ids.astype(i32), jnp.full((N_PAD - N,), G, i32)]).reshape(N_PAD, 1)
    ones_pat = (jnp.zeros((2, CHUNK, 16), f32)
                .at[0, :, 0].set(1.0).at[1, :, 1].set(1.0))

    deg = _sc_degrees(edges3, ones_pat, jnp.zeros((N_PAD, 16), f32))
    w1x, q = _tc_pre(x_pad, W1_0, W2_0, b_0.reshape(1, 64))
    a, b, p = _tc_prep(deg[0], deg[1], w1x)

    layers = [
        (64, 32, W1_1, W2_1, b_1),
        (32, 16, W1_2, W2_2, b_2),
        (16, 16, W1_3, W2_3, b_3),
    ]
    for (dout, dnext, W1n, W2n, biasn) in layers:
        acc = _sc_scatter(p, edges3, jnp.zeros((N_PAD, dout), f32), dout)
        p, q = _tc_post(acc[0], acc[1], b, q, a, W1n, W2n,
                        biasn.reshape(1, dnext), dout, dnext)
    acc = _sc_scatter(p, edges3, jnp.zeros((N_PAD, 16), f32), 16)
    return _tc_final(acc[0], acc[1], b, q, seg2, Wo, bo.reshape(1, 1))


# speedup vs baseline: 1.0041x; 1.0041x over previous
import functools

import jax
import jax.numpy as jnp
from jax import lax
from jax.experimental import pallas as pl
from jax.experimental.pallas import tpu as pltpu
from jax.experimental.pallas import tpu_sc as plsc

N = 10000
E = 160000
G = 64
N_PAD = 10240
E_PAD = 163840
CHUNK = 128
NW = 32
NSUB = 16
EP_W = E_PAD // NW
NCHUNK = EP_W // CHUNK
ROWS_W = N_PAD // NSUB
BN = 2048
NBLK = N_PAD // BN


def _sc_mesh():
    return plsc.VectorSubcoreMesh(core_axis_name="c", subcore_axis_name="s",
                                  num_cores=2, num_subcores=NSUB)


K_PIPE = 10


def _sc_degrees(edges3, ones_pat, zeros16):
    @functools.partial(
        pl.kernel,
        out_type=jax.ShapeDtypeStruct((2, N_PAD, 16), jnp.float32),
        mesh=_sc_mesh(),
        scratch_types=[
            pltpu.VMEM((K_PIPE, 2, CHUNK), jnp.int32),
            pltpu.VMEM((CHUNK, 16), jnp.float32),
            pltpu.VMEM((CHUNK, 16), jnp.float32),
            pltpu.SemaphoreType.DMA,
            pltpu.SemaphoreType.DMA,
            pltpu.VMEM_SHARED((N_PAD, 16), jnp.float32),
        ],
    )
    def k(ed_ref, ones_ref, zero_ref, out_ref, idx, e_src, e_dst, sem_i,
          sem_sc, acc):
        cid = lax.axis_index("c")
        sid = lax.axis_index("s")
        cbase = (cid * NSUB + sid) * NCHUNK
        r0 = sid * ROWS_W
        pltpu.sync_copy(ones_ref.at[0], e_src)
        pltpu.sync_copy(ones_ref.at[1], e_dst)
        pltpu.sync_copy(zero_ref.at[pl.ds(r0, ROWS_W)], acc.at[pl.ds(r0, ROWS_W)])
        plsc.subcore_barrier()

        def body(j, carry):
            c0 = cbase + j * K_PIPE
            pltpu.async_copy(ed_ref.at[pl.ds(c0, K_PIPE)], idx, sem_i).wait()
            scats = []
            for s in range(K_PIPE):
                scats.append(pltpu.async_copy(
                    e_src, acc.at[idx.at[s, 0]], sem_sc, add=True))
                scats.append(pltpu.async_copy(
                    e_dst, acc.at[idx.at[s, 1]], sem_sc, add=True))
            for d in scats:
                d.wait()
            return carry

        lax.fori_loop(0, NCHUNK // K_PIPE, body, 0)
        plsc.subcore_barrier()
        pltpu.sync_copy(acc.at[pl.ds(r0, ROWS_W)], out_ref.at[cid, pl.ds(r0, ROWS_W)])

    return k(edges3, ones_pat, zeros16)


def _sc_scatter(p_hbm, edges3, zeros, dout):
    @functools.partial(
        pl.kernel,
        out_type=jax.ShapeDtypeStruct((2, N_PAD, dout), jnp.float32),
        mesh=_sc_mesh(),
        compiler_params=pltpu.CompilerParams(use_tc_tiling_on_sc=False),
        scratch_types=[
            pltpu.VMEM((K_PIPE, 2, CHUNK), jnp.int32),
            [pltpu.VMEM((CHUNK, dout), jnp.float32)] * K_PIPE,
            pltpu.SemaphoreType.DMA,
            [pltpu.SemaphoreType.DMA] * K_PIPE,
            pltpu.SemaphoreType.DMA,
            pltpu.VMEM_SHARED((N_PAD, dout), jnp.float32),
        ],
    )
    def k(p_ref, ed_ref, zero_ref, out_ref, idx, rows, sem_i, sem_g, sem_sc,
          acc):
        cid = lax.axis_index("c")
        sid = lax.axis_index("s")
        cbase = (cid * NSUB + sid) * NCHUNK
        r0 = sid * ROWS_W
        pltpu.sync_copy(zero_ref.at[pl.ds(r0, ROWS_W)], acc.at[pl.ds(r0, ROWS_W)])
        plsc.subcore_barrier()

        def body(j, carry):
            c0 = cbase + j * K_PIPE
            pltpu.async_copy(ed_ref.at[pl.ds(c0, K_PIPE)], idx, sem_i).wait()
            gathers = [
                pltpu.async_copy(p_ref.at[idx.at[s, 0]], rows[s], sem_g[s])
                for s in range(K_PIPE)
            ]
            scats = []
            for s in range(K_PIPE):
                gathers[s].wait()
                scats.append(pltpu.async_copy(
                    rows[s], acc.at[idx.at[s, 1]], sem_sc, add=True))
            for d in scats:
                d.wait()
            return carry

        lax.fori_loop(0, NCHUNK // K_PIPE, body, 0)
        plsc.subcore_barrier()
        pltpu.sync_copy(acc.at[pl.ds(r0, ROWS_W)], out_ref.at[cid, pl.ds(r0, ROWS_W)])

    return k(p_hbm, edges3, zeros)


def _tc_pre(x_pad, W1_0, W2_0, b_0):
    def body(x_ref, w1_ref, w2_ref, bias_ref, w1x_ref, q_ref):
        xb = x_ref[...]
        w1x_ref[...] = jnp.dot(xb, w1_ref[...],
                               preferred_element_type=jnp.float32)
        q_ref[...] = jnp.dot(xb, w2_ref[...],
                             preferred_element_type=jnp.float32) + bias_ref[...]

    return pl.pallas_call(
        body,
        grid=(NBLK,),
        in_specs=[
            pl.BlockSpec((BN, 256), lambda i: (i, 0)),
            pl.BlockSpec((256, 64), lambda i: (0, 0)),
            pl.BlockSpec((256, 64), lambda i: (0, 0)),
            pl.BlockSpec((1, 64), lambda i: (0, 0)),
        ],
        out_specs=[
            pl.BlockSpec((BN, 64), lambda i: (i, 0)),
            pl.BlockSpec((BN, 64), lambda i: (i, 0)),
        ],
        out_shape=[
            jax.ShapeDtypeStruct((N_PAD, 64), jnp.float32),
            jax.ShapeDtypeStruct((N_PAD, 64), jnp.float32),
        ],
    )(x_pad, W1_0, W2_0, b_0)


def _tc_prep(d0, d1, w1x):
    def body(d0_ref, d1_ref, w1x_ref, a_ref, b_ref, p_ref):
        deg = d0_ref[...] + d1_ref[...]
        a = lax.rsqrt(jnp.maximum(deg[:, 0:1], 1.0))
        b = lax.rsqrt(jnp.maximum(deg[:, 1:2], 1.0))
        a_ref[...] = a
        b_ref[...] = b
        p_ref[...] = w1x_ref[...] * a

    return pl.pallas_call(
        body,
        grid=(NBLK,),
        in_specs=[
            pl.BlockSpec((BN, 16), lambda i: (i, 0)),
            pl.BlockSpec((BN, 16), lambda i: (i, 0)),
            pl.BlockSpec((BN, 64), lambda i: (i, 0)),
        ],
        out_specs=[
            pl.BlockSpec((BN, 1), lambda i: (i, 0)),
            pl.BlockSpec((BN, 1), lambda i: (i, 0)),
            pl.BlockSpec((BN, 64), lambda i: (i, 0)),
        ],
        out_shape=[
            jax.ShapeDtypeStruct((N_PAD, 1), jnp.float32),
            jax.ShapeDtypeStruct((N_PAD, 1), jnp.float32),
            jax.ShapeDtypeStruct((N_PAD, 64), jnp.float32),
        ],
    )(d0, d1, w1x)


def _tc_post(acc0, acc1, bvec, q, avec, W1n, W2n, biasn, dout, dnext):
    def body(a0_ref, a1_ref, b_ref, q_ref, a_ref, w1n_ref, w2n_ref, bias_ref,
             p_ref, qn_ref):
        i = pl.program_id(0)
        rows = lax.broadcasted_iota(jnp.int32, (BN, 1), 0) + i * BN
        msk = (rows < N).astype(jnp.float32)
        agg = (a0_ref[...] + a1_ref[...]) * b_ref[...]
        h = jnp.maximum(agg + q_ref[...], 0.0) * msk
        p_ref[...] = jnp.dot(h, w1n_ref[...],
                             preferred_element_type=jnp.float32) * a_ref[...]
        qn_ref[...] = jnp.dot(h, w2n_ref[...],
                              preferred_element_type=jnp.float32) + bias_ref[...]

    return pl.pallas_call(
        body,
        grid=(NBLK,),
        in_specs=[
            pl.BlockSpec((BN, dout), lambda i: (i, 0)),
            pl.BlockSpec((BN, dout), lambda i: (i, 0)),
            pl.BlockSpec((BN, 1), lambda i: (i, 0)),
            pl.BlockSpec((BN, dout), lambda i: (i, 0)),
            pl.BlockSpec((BN, 1), lambda i: (i, 0)),
            pl.BlockSpec((dout, dnext), lambda i: (0, 0)),
            pl.BlockSpec((dout, dnext), lambda i: (0, 0)),
            pl.BlockSpec((1, dnext), lambda i: (0, 0)),
        ],
        out_specs=[
            pl.BlockSpec((BN, dnext), lambda i: (i, 0)),
            pl.BlockSpec((BN, dnext), lambda i: (i, 0)),
        ],
        out_shape=[
            jax.ShapeDtypeStruct((N_PAD, dnext), jnp.float32),
            jax.ShapeDtypeStruct((N_PAD, dnext), jnp.float32),
        ],
    )(acc0, acc1, bvec, q, avec, W1n, W2n, biasn)


def _tc_final(acc0, acc1, bvec, q3, seg2, Wo, bo):
    def body(a0_ref, a1_ref, b_ref, q_ref, seg_ref, wo_ref,
             bo_ref, out_ref, sums, counts):
        i = pl.program_id(0)

        @pl.when(i == 0)
        def _():
            sums[...] = jnp.zeros_like(sums)
            counts[...] = jnp.zeros_like(counts)

        rows = lax.broadcasted_iota(jnp.int32, (BN, 1), 0) + i * BN
        msk = (rows < N).astype(jnp.float32)
        agg = (a0_ref[...] + a1_ref[...]) * b_ref[...]
        h = jnp.maximum(agg + q_ref[...], 0.0) * msk
        gid = lax.broadcasted_iota(jnp.int32, (BN, G), 1)
        onehot = (seg_ref[...] == gid).astype(jnp.float32)
        sums[...] += lax.dot_general(onehot, h, (((0,), (0,)), ((), ())),
                                     preferred_element_type=jnp.float32)
        counts[...] += lax.dot_general(onehot, jnp.ones((BN, 16), jnp.float32),
                                       (((0,), (0,)), ((), ())),
                                       preferred_element_type=jnp.float32)

        @pl.when(i == NBLK - 1)
        def _():
            pooled = sums[...] / jnp.maximum(counts[...], 1.0)
            out_ref[...] = jnp.dot(pooled, wo_ref[...],
                                   preferred_element_type=jnp.float32) + bo_ref[...]

    return pl.pallas_call(
        body,
        grid=(NBLK,),
        in_specs=[
            pl.BlockSpec((BN, 16), lambda i: (i, 0)),
            pl.BlockSpec((BN, 16), lambda i: (i, 0)),
            pl.BlockSpec((BN, 1), lambda i: (i, 0)),
            pl.BlockSpec((BN, 16), lambda i: (i, 0)),
            pl.BlockSpec((BN, 1), lambda i: (i, 0)),
            pl.BlockSpec((16, 1), lambda i: (0, 0)),
            pl.BlockSpec((1, 1), lambda i: (0, 0)),
        ],
        out_specs=pl.BlockSpec((G, 1), lambda i: (0, 0)),
        out_shape=jax.ShapeDtypeStruct((G, 1), jnp.float32),
        scratch_shapes=[
            pltpu.VMEM((G, 16), jnp.float32),
            pltpu.VMEM((G, 16), jnp.float32),
        ],
    )(acc0, acc1, bvec, q3, seg2, Wo, bo)


def kernel(x, edge_index, segment_ids, W1_0, W2_0, b_0, W1_1, W2_1, b_1,
           W1_2, W2_2, b_2, W1_3, W2_3, b_3, Wo, bo):
    f32 = jnp.float32
    i32 = jnp.int32
    pad_e = jnp.full((2, E_PAD - E), N, i32)
    edges3 = jnp.concatenate([edge_index.astype(i32), pad_e], axis=1) \
        .reshape(2, E_PAD // CHUNK, CHUNK).transpose(1, 0, 2)
    x_pad = jnp.pad(x, ((0, N_PAD - N), (0, 0)))
    seg2 = jnp.concatenate(
        [segment_ids.astype(i32), jnp.full((N_PAD - N,), G, i32)]).reshape(N_PAD, 1)
    ones_pat = (jnp.zeros((2, CHUNK, 16), f32)
                .at[0, :, 0].set(1.0).at[1, :, 1].set(1.0))

    deg = _sc_degrees(edges3, ones_pat, jnp.zeros((N_PAD, 16), f32))
    w1x, q = _tc_pre(x_pad, W1_0, W2_0, b_0.reshape(1, 64))
    a, b, p = _tc_prep(deg[0], deg[1], w1x)

    layers = [
        (64, 32, W1_1, W2_1, b_1),
        (32, 16, W1_2, W2_2, b_2),
        (16, 16, W1_3, W2_3, b_3),
    ]
    for (dout, dnext, W1n, W2n, biasn) in layers:
        acc = _sc_scatter(p, edges3, jnp.zeros((N_PAD, dout), f32), dout)
        p, q = _tc_post(acc[0], acc[1], b, q, a, W1n, W2n,
                        biasn.reshape(1, dnext), dout, dnext)
    acc = _sc_scatter(p, edges3, jnp.zeros((N_PAD, 16), f32), 16)
    return _tc_final(acc[0], acc[1], b, q, seg2, Wo, bo.reshape(1, 1))


# speedup vs baseline: 1.0154x; 1.0112x over previous
import functools

import jax
import jax.numpy as jnp
from jax import lax
from jax.experimental import pallas as pl
from jax.experimental.pallas import tpu as pltpu
from jax.experimental.pallas import tpu_sc as plsc

N = 10000
E = 160000
G = 64
N_PAD = 10240
E_PAD = 163840
CHUNK = 128
NW = 32
NSUB = 16
EP_W = E_PAD // NW
NCHUNK = EP_W // CHUNK
ROWS_W = N_PAD // NSUB
BN = 2048
NBLK = N_PAD // BN


def _sc_mesh():
    return plsc.VectorSubcoreMesh(core_axis_name="c", subcore_axis_name="s",
                                  num_cores=2, num_subcores=NSUB)


K_PIPE = 10


def _sc_degrees(edges3, ones_pat, zeros16):
    @functools.partial(
        pl.kernel,
        out_type=jax.ShapeDtypeStruct((2, N_PAD, 16), jnp.float32),
        mesh=_sc_mesh(),
        scratch_types=[
            pltpu.VMEM((K_PIPE, 2, CHUNK), jnp.int32),
            pltpu.VMEM((CHUNK, 16), jnp.float32),
            pltpu.VMEM((CHUNK, 16), jnp.float32),
            pltpu.SemaphoreType.DMA,
            pltpu.SemaphoreType.DMA,
            pltpu.VMEM_SHARED((N_PAD, 16), jnp.float32),
        ],
    )
    def k(ed_ref, ones_ref, zero_ref, out_ref, idx, e_src, e_dst, sem_i,
          sem_sc, acc):
        cid = lax.axis_index("c")
        sid = lax.axis_index("s")
        cbase = (cid * NSUB + sid) * NCHUNK
        r0 = sid * ROWS_W
        pltpu.sync_copy(ones_ref.at[0], e_src)
        pltpu.sync_copy(ones_ref.at[1], e_dst)
        pltpu.sync_copy(zero_ref.at[pl.ds(r0, ROWS_W)], acc.at[pl.ds(r0, ROWS_W)])
        plsc.subcore_barrier()

        def body(j, carry):
            c0 = cbase + j * K_PIPE
            pltpu.async_copy(ed_ref.at[pl.ds(c0, K_PIPE)], idx, sem_i).wait()
            scats = []
            for s in range(K_PIPE):
                scats.append(pltpu.async_copy(
                    e_src, acc.at[idx.at[s, 0]], sem_sc, add=True))
                scats.append(pltpu.async_copy(
                    e_dst, acc.at[idx.at[s, 1]], sem_sc, add=True))
            for d in scats:
                d.wait()
            return carry

        lax.fori_loop(0, NCHUNK // K_PIPE, body, 0)
        plsc.subcore_barrier()
        pltpu.sync_copy(acc.at[pl.ds(r0, ROWS_W)], out_ref.at[cid, pl.ds(r0, ROWS_W)])

    return k(edges3, ones_pat, zeros16)


def _sc_scatter(p_hbm, edges3, zeros, dout):
    kp = K_PIPE if dout >= 64 else 2 * K_PIPE

    @functools.partial(
        pl.kernel,
        out_type=jax.ShapeDtypeStruct((2, N_PAD, dout), jnp.float32),
        mesh=_sc_mesh(),
        compiler_params=pltpu.CompilerParams(use_tc_tiling_on_sc=False),
        scratch_types=[
            pltpu.VMEM((kp, 2, CHUNK), jnp.int32),
            [pltpu.VMEM((CHUNK, dout), jnp.float32)] * kp,
            pltpu.SemaphoreType.DMA,
            [pltpu.SemaphoreType.DMA] * kp,
            pltpu.SemaphoreType.DMA,
            pltpu.VMEM_SHARED((N_PAD, dout), jnp.float32),
        ],
    )
    def k(p_ref, ed_ref, zero_ref, out_ref, idx, rows, sem_i, sem_g, sem_sc,
          acc):
        cid = lax.axis_index("c")
        sid = lax.axis_index("s")
        cbase = (cid * NSUB + sid) * NCHUNK
        r0 = sid * ROWS_W
        pltpu.sync_copy(zero_ref.at[pl.ds(r0, ROWS_W)], acc.at[pl.ds(r0, ROWS_W)])
        plsc.subcore_barrier()

        def body(j, carry):
            c0 = cbase + j * kp
            pltpu.async_copy(ed_ref.at[pl.ds(c0, kp)], idx, sem_i).wait()
            gathers = [
                pltpu.async_copy(p_ref.at[idx.at[s, 0]], rows[s], sem_g[s])
                for s in range(kp)
            ]
            scats = []
            for s in range(kp):
                gathers[s].wait()
                scats.append(pltpu.async_copy(
                    rows[s], acc.at[idx.at[s, 1]], sem_sc, add=True))
            for d in scats:
                d.wait()
            return carry

        lax.fori_loop(0, NCHUNK // kp, body, 0)
        plsc.subcore_barrier()
        pltpu.sync_copy(acc.at[pl.ds(r0, ROWS_W)], out_ref.at[cid, pl.ds(r0, ROWS_W)])

    return k(p_hbm, edges3, zeros)


def _tc_pre(x_pad, W1_0, W2_0, b_0):
    def body(x_ref, w1_ref, w2_ref, bias_ref, w1x_ref, q_ref):
        xb = x_ref[...]
        w1x_ref[...] = jnp.dot(xb, w1_ref[...],
                               preferred_element_type=jnp.float32)
        q_ref[...] = jnp.dot(xb, w2_ref[...],
                             preferred_element_type=jnp.float32) + bias_ref[...]

    return pl.pallas_call(
        body,
        grid=(NBLK,),
        in_specs=[
            pl.BlockSpec((BN, 256), lambda i: (i, 0)),
            pl.BlockSpec((256, 64), lambda i: (0, 0)),
            pl.BlockSpec((256, 64), lambda i: (0, 0)),
            pl.BlockSpec((1, 64), lambda i: (0, 0)),
        ],
        out_specs=[
            pl.BlockSpec((BN, 64), lambda i: (i, 0)),
            pl.BlockSpec((BN, 64), lambda i: (i, 0)),
        ],
        out_shape=[
            jax.ShapeDtypeStruct((N_PAD, 64), jnp.float32),
            jax.ShapeDtypeStruct((N_PAD, 64), jnp.float32),
        ],
    )(x_pad, W1_0, W2_0, b_0)


def _tc_prep(d0, d1, w1x):
    def body(d0_ref, d1_ref, w1x_ref, a_ref, b_ref, p_ref):
        deg = d0_ref[...] + d1_ref[...]
        a = lax.rsqrt(jnp.maximum(deg[:, 0:1], 1.0))
        b = lax.rsqrt(jnp.maximum(deg[:, 1:2], 1.0))
        a_ref[...] = a
        b_ref[...] = b
        p_ref[...] = w1x_ref[...] * a

    return pl.pallas_call(
        body,
        grid=(NBLK,),
        in_specs=[
            pl.BlockSpec((BN, 16), lambda i: (i, 0)),
            pl.BlockSpec((BN, 16), lambda i: (i, 0)),
            pl.BlockSpec((BN, 64), lambda i: (i, 0)),
        ],
        out_specs=[
            pl.BlockSpec((BN, 1), lambda i: (i, 0)),
            pl.BlockSpec((BN, 1), lambda i: (i, 0)),
            pl.BlockSpec((BN, 64), lambda i: (i, 0)),
        ],
        out_shape=[
            jax.ShapeDtypeStruct((N_PAD, 1), jnp.float32),
            jax.ShapeDtypeStruct((N_PAD, 1), jnp.float32),
            jax.ShapeDtypeStruct((N_PAD, 64), jnp.float32),
        ],
    )(d0, d1, w1x)


def _tc_post(acc0, acc1, bvec, q, avec, W1n, W2n, biasn, dout, dnext):
    def body(a0_ref, a1_ref, b_ref, q_ref, a_ref, w1n_ref, w2n_ref, bias_ref,
             p_ref, qn_ref):
        i = pl.program_id(0)
        rows = lax.broadcasted_iota(jnp.int32, (BN, 1), 0) + i * BN
        msk = (rows < N).astype(jnp.float32)
        agg = (a0_ref[...] + a1_ref[...]) * b_ref[...]
        h = jnp.maximum(agg + q_ref[...], 0.0) * msk
        p_ref[...] = jnp.dot(h, w1n_ref[...],
                             preferred_element_type=jnp.float32) * a_ref[...]
        qn_ref[...] = jnp.dot(h, w2n_ref[...],
                              preferred_element_type=jnp.float32) + bias_ref[...]

    return pl.pallas_call(
        body,
        grid=(NBLK,),
        in_specs=[
            pl.BlockSpec((BN, dout), lambda i: (i, 0)),
            pl.BlockSpec((BN, dout), lambda i: (i, 0)),
            pl.BlockSpec((BN, 1), lambda i: (i, 0)),
            pl.BlockSpec((BN, dout), lambda i: (i, 0)),
            pl.BlockSpec((BN, 1), lambda i: (i, 0)),
            pl.BlockSpec((dout, dnext), lambda i: (0, 0)),
            pl.BlockSpec((dout, dnext), lambda i: (0, 0)),
            pl.BlockSpec((1, dnext), lambda i: (0, 0)),
        ],
        out_specs=[
            pl.BlockSpec((BN, dnext), lambda i: (i, 0)),
            pl.BlockSpec((BN, dnext), lambda i: (i, 0)),
        ],
        out_shape=[
            jax.ShapeDtypeStruct((N_PAD, dnext), jnp.float32),
            jax.ShapeDtypeStruct((N_PAD, dnext), jnp.float32),
        ],
    )(acc0, acc1, bvec, q, avec, W1n, W2n, biasn)


def _tc_final(acc0, acc1, bvec, q3, seg2, Wo, bo):
    def body(a0_ref, a1_ref, b_ref, q_ref, seg_ref, wo_ref,
             bo_ref, out_ref, sums, counts):
        i = pl.program_id(0)

        @pl.when(i == 0)
        def _():
            sums[...] = jnp.zeros_like(sums)
            counts[...] = jnp.zeros_like(counts)

        rows = lax.broadcasted_iota(jnp.int32, (BN, 1), 0) + i * BN
        msk = (rows < N).astype(jnp.float32)
        agg = (a0_ref[...] + a1_ref[...]) * b_ref[...]
        h = jnp.maximum(agg + q_ref[...], 0.0) * msk
        gid = lax.broadcasted_iota(jnp.int32, (BN, G), 1)
        onehot = (seg_ref[...] == gid).astype(jnp.float32)
        sums[...] += lax.dot_general(onehot, h, (((0,), (0,)), ((), ())),
                                     preferred_element_type=jnp.float32)
        counts[...] += lax.dot_general(onehot, jnp.ones((BN, 16), jnp.float32),
                                       (((0,), (0,)), ((), ())),
                                       preferred_element_type=jnp.float32)

        @pl.when(i == NBLK - 1)
        def _():
            pooled = sums[...] / jnp.maximum(counts[...], 1.0)
            out_ref[...] = jnp.dot(pooled, wo_ref[...],
                                   preferred_element_type=jnp.float32) + bo_ref[...]

    return pl.pallas_call(
        body,
        grid=(NBLK,),
        in_specs=[
            pl.BlockSpec((BN, 16), lambda i: (i, 0)),
            pl.BlockSpec((BN, 16), lambda i: (i, 0)),
            pl.BlockSpec((BN, 1), lambda i: (i, 0)),
            pl.BlockSpec((BN, 16), lambda i: (i, 0)),
            pl.BlockSpec((BN, 1), lambda i: (i, 0)),
            pl.BlockSpec((16, 1), lambda i: (0, 0)),
            pl.BlockSpec((1, 1), lambda i: (0, 0)),
        ],
        out_specs=pl.BlockSpec((G, 1), lambda i: (0, 0)),
        out_shape=jax.ShapeDtypeStruct((G, 1), jnp.float32),
        scratch_shapes=[
            pltpu.VMEM((G, 16), jnp.float32),
            pltpu.VMEM((G, 16), jnp.float32),
        ],
    )(acc0, acc1, bvec, q3, seg2, Wo, bo)


def kernel(x, edge_index, segment_ids, W1_0, W2_0, b_0, W1_1, W2_1, b_1,
           W1_2, W2_2, b_2, W1_3, W2_3, b_3, Wo, bo):
    f32 = jnp.float32
    i32 = jnp.int32
    pad_e = jnp.full((2, E_PAD - E), N, i32)
    edges3 = jnp.concatenate([edge_index.astype(i32), pad_e], axis=1) \
        .reshape(2, E_PAD // CHUNK, CHUNK).transpose(1, 0, 2)
    x_pad = jnp.pad(x, ((0, N_PAD - N), (0, 0)))
    seg2 = jnp.concatenate(
        [segment_ids.astype(i32), jnp.full((N_PAD - N,), G, i32)]).reshape(N_PAD, 1)
    ones_pat = (jnp.zeros((2, CHUNK, 16), f32)
                .at[0, :, 0].set(1.0).at[1, :, 1].set(1.0))

    deg = _sc_degrees(edges3, ones_pat, jnp.zeros((N_PAD, 16), f32))
    w1x, q = _tc_pre(x_pad, W1_0, W2_0, b_0.reshape(1, 64))
    a, b, p = _tc_prep(deg[0], deg[1], w1x)

    layers = [
        (64, 32, W1_1, W2_1, b_1),
        (32, 16, W1_2, W2_2, b_2),
        (16, 16, W1_3, W2_3, b_3),
    ]
    for (dout, dnext, W1n, W2n, biasn) in layers:
        acc = _sc_scatter(p, edges3, jnp.zeros((N_PAD, dout), f32), dout)
        p, q = _tc_post(acc[0], acc[1], b, q, a, W1n, W2n,
                        biasn.reshape(1, dnext), dout, dnext)
    acc = _sc_scatter(p, edges3, jnp.zeros((N_PAD, 16), f32), 16)
    return _tc_final(acc[0], acc[1], b, q, seg2, Wo, bo.reshape(1, 1))


# speedup vs baseline: 1.1985x; 1.1804x over previous
import functools

import jax
import jax.numpy as jnp
from jax import lax
from jax.experimental import pallas as pl
from jax.experimental.pallas import tpu as pltpu
from jax.experimental.pallas import tpu_sc as plsc

N = 10000
E = 160000
G = 64
N_PAD = 10240
E_PAD = 163840
CHUNK = 128
NW = 32
NSUB = 16
EP_W = E_PAD // NW
NCHUNK = EP_W // CHUNK
ROWS_W = N_PAD // NSUB
BN = 2048
NBLK = N_PAD // BN


def _sc_mesh():
    return plsc.VectorSubcoreMesh(core_axis_name="c", subcore_axis_name="s",
                                  num_cores=2, num_subcores=NSUB)


K_PIPE = 10


def _sc_degrees(edges3, ones_pat, zeros16):
    @functools.partial(
        pl.kernel,
        out_type=jax.ShapeDtypeStruct((2, N_PAD, 16), jnp.float32),
        mesh=_sc_mesh(),
        scratch_types=[
            pltpu.VMEM((K_PIPE, 2, CHUNK), jnp.int32),
            pltpu.VMEM((CHUNK, 16), jnp.float32),
            pltpu.VMEM((CHUNK, 16), jnp.float32),
            pltpu.SemaphoreType.DMA,
            pltpu.SemaphoreType.DMA,
            pltpu.VMEM_SHARED((N_PAD, 16), jnp.float32),
        ],
    )
    def k(ed_ref, ones_ref, zero_ref, out_ref, idx, e_src, e_dst, sem_i,
          sem_sc, acc):
        cid = lax.axis_index("c")
        sid = lax.axis_index("s")
        cbase = (cid * NSUB + sid) * NCHUNK
        r0 = sid * ROWS_W
        pltpu.sync_copy(ones_ref.at[0], e_src)
        pltpu.sync_copy(ones_ref.at[1], e_dst)
        pltpu.sync_copy(zero_ref.at[pl.ds(r0, ROWS_W)], acc.at[pl.ds(r0, ROWS_W)])
        plsc.subcore_barrier()

        def body(j, carry):
            c0 = cbase + j * K_PIPE
            pltpu.async_copy(ed_ref.at[pl.ds(c0, K_PIPE)], idx, sem_i).wait()
            scats = []
            for s in range(K_PIPE):
                scats.append(pltpu.async_copy(
                    e_src, acc.at[idx.at[s, 0]], sem_sc, add=True))
                scats.append(pltpu.async_copy(
                    e_dst, acc.at[idx.at[s, 1]], sem_sc, add=True))
            for d in scats:
                d.wait()
            return carry

        lax.fori_loop(0, NCHUNK // K_PIPE, body, 0)
        plsc.subcore_barrier()
        pltpu.sync_copy(acc.at[pl.ds(r0, ROWS_W)], out_ref.at[cid, pl.ds(r0, ROWS_W)])

    return k(edges3, ones_pat, zeros16)


def _sc_scatter(p_hbm, edges3, zeros, dout):
    kp = K_PIPE if dout >= 64 else 2 * K_PIPE
    stage_p = dout < 64
    stage_scratch = ([pltpu.VMEM_SHARED((N_PAD, dout), jnp.float32)]
                     if stage_p else [])

    @functools.partial(
        pl.kernel,
        out_type=jax.ShapeDtypeStruct((2, N_PAD, dout), jnp.float32),
        mesh=_sc_mesh(),
        compiler_params=pltpu.CompilerParams(use_tc_tiling_on_sc=False),
        scratch_types=[
            pltpu.VMEM((kp, 2, CHUNK), jnp.int32),
            [pltpu.VMEM((CHUNK, dout), jnp.float32)] * kp,
            pltpu.SemaphoreType.DMA,
            [pltpu.SemaphoreType.DMA] * kp,
            pltpu.SemaphoreType.DMA,
            pltpu.VMEM_SHARED((N_PAD, dout), jnp.float32),
        ] + stage_scratch,
    )
    def k(p_ref, ed_ref, zero_ref, out_ref, idx, rows, sem_i, sem_g, sem_sc,
          acc, *maybe_psh):
        cid = lax.axis_index("c")
        sid = lax.axis_index("s")
        cbase = (cid * NSUB + sid) * NCHUNK
        r0 = sid * ROWS_W
        if stage_p:
            p_src = maybe_psh[0]
            pltpu.sync_copy(p_ref.at[pl.ds(r0, ROWS_W)],
                            p_src.at[pl.ds(r0, ROWS_W)])
        else:
            p_src = p_ref
        pltpu.sync_copy(zero_ref.at[pl.ds(r0, ROWS_W)], acc.at[pl.ds(r0, ROWS_W)])
        plsc.subcore_barrier()

        def body(j, carry):
            c0 = cbase + j * kp
            pltpu.async_copy(ed_ref.at[pl.ds(c0, kp)], idx, sem_i).wait()
            gathers = [
                pltpu.async_copy(p_src.at[idx.at[s, 0]], rows[s], sem_g[s])
                for s in range(kp)
            ]
            scats = []
            for s in range(kp):
                gathers[s].wait()
                scats.append(pltpu.async_copy(
                    rows[s], acc.at[idx.at[s, 1]], sem_sc, add=True))
            for d in scats:
                d.wait()
            return carry

        lax.fori_loop(0, NCHUNK // kp, body, 0)
        plsc.subcore_barrier()
        pltpu.sync_copy(acc.at[pl.ds(r0, ROWS_W)], out_ref.at[cid, pl.ds(r0, ROWS_W)])

    return k(p_hbm, edges3, zeros)


def _tc_pre(x_pad, W1_0, W2_0, b_0):
    def body(x_ref, w1_ref, w2_ref, bias_ref, w1x_ref, q_ref):
        xb = x_ref[...]
        w1x_ref[...] = jnp.dot(xb, w1_ref[...],
                               preferred_element_type=jnp.float32)
        q_ref[...] = jnp.dot(xb, w2_ref[...],
                             preferred_element_type=jnp.float32) + bias_ref[...]

    return pl.pallas_call(
        body,
        grid=(NBLK,),
        in_specs=[
            pl.BlockSpec((BN, 256), lambda i: (i, 0)),
            pl.BlockSpec((256, 64), lambda i: (0, 0)),
            pl.BlockSpec((256, 64), lambda i: (0, 0)),
            pl.BlockSpec((1, 64), lambda i: (0, 0)),
        ],
        out_specs=[
            pl.BlockSpec((BN, 64), lambda i: (i, 0)),
            pl.BlockSpec((BN, 64), lambda i: (i, 0)),
        ],
        out_shape=[
            jax.ShapeDtypeStruct((N_PAD, 64), jnp.float32),
            jax.ShapeDtypeStruct((N_PAD, 64), jnp.float32),
        ],
    )(x_pad, W1_0, W2_0, b_0)


def _tc_prep(d0, d1, w1x):
    def body(d0_ref, d1_ref, w1x_ref, a_ref, b_ref, p_ref):
        deg = d0_ref[...] + d1_ref[...]
        a = lax.rsqrt(jnp.maximum(deg[:, 0:1], 1.0))
        b = lax.rsqrt(jnp.maximum(deg[:, 1:2], 1.0))
        a_ref[...] = a
        b_ref[...] = b
        p_ref[...] = w1x_ref[...] * a

    return pl.pallas_call(
        body,
        grid=(NBLK,),
        in_specs=[
            pl.BlockSpec((BN, 16), lambda i: (i, 0)),
            pl.BlockSpec((BN, 16), lambda i: (i, 0)),
            pl.BlockSpec((BN, 64), lambda i: (i, 0)),
        ],
        out_specs=[
            pl.BlockSpec((BN, 1), lambda i: (i, 0)),
            pl.BlockSpec((BN, 1), lambda i: (i, 0)),
            pl.BlockSpec((BN, 64), lambda i: (i, 0)),
        ],
        out_shape=[
            jax.ShapeDtypeStruct((N_PAD, 1), jnp.float32),
            jax.ShapeDtypeStruct((N_PAD, 1), jnp.float32),
            jax.ShapeDtypeStruct((N_PAD, 64), jnp.float32),
        ],
    )(d0, d1, w1x)


def _tc_post(acc0, acc1, bvec, q, avec, W1n, W2n, biasn, dout, dnext):
    def body(a0_ref, a1_ref, b_ref, q_ref, a_ref, w1n_ref, w2n_ref, bias_ref,
             p_ref, qn_ref):
        i = pl.program_id(0)
        rows = lax.broadcasted_iota(jnp.int32, (BN, 1), 0) + i * BN
        msk = (rows < N).astype(jnp.float32)
        agg = (a0_ref[...] + a1_ref[...]) * b_ref[...]
        h = jnp.maximum(agg + q_ref[...], 0.0) * msk
        p_ref[...] = jnp.dot(h, w1n_ref[...],
                             preferred_element_type=jnp.float32) * a_ref[...]
        qn_ref[...] = jnp.dot(h, w2n_ref[...],
                              preferred_element_type=jnp.float32) + bias_ref[...]

    return pl.pallas_call(
        body,
        grid=(NBLK,),
        in_specs=[
            pl.BlockSpec((BN, dout), lambda i: (i, 0)),
            pl.BlockSpec((BN, dout), lambda i: (i, 0)),
            pl.BlockSpec((BN, 1), lambda i: (i, 0)),
            pl.BlockSpec((BN, dout), lambda i: (i, 0)),
            pl.BlockSpec((BN, 1), lambda i: (i, 0)),
            pl.BlockSpec((dout, dnext), lambda i: (0, 0)),
            pl.BlockSpec((dout, dnext), lambda i: (0, 0)),
            pl.BlockSpec((1, dnext), lambda i: (0, 0)),
        ],
        out_specs=[
            pl.BlockSpec((BN, dnext), lambda i: (i, 0)),
            pl.BlockSpec((BN, dnext), lambda i: (i, 0)),
        ],
        out_shape=[
            jax.ShapeDtypeStruct((N_PAD, dnext), jnp.float32),
            jax.ShapeDtypeStruct((N_PAD, dnext), jnp.float32),
        ],
    )(acc0, acc1, bvec, q, avec, W1n, W2n, biasn)


def _tc_final(acc0, acc1, bvec, q3, seg2, Wo, bo):
    def body(a0_ref, a1_ref, b_ref, q_ref, seg_ref, wo_ref,
             bo_ref, out_ref, sums, counts):
        i = pl.program_id(0)

        @pl.when(i == 0)
        def _():
            sums[...] = jnp.zeros_like(sums)
            counts[...] = jnp.zeros_like(counts)

        rows = lax.broadcasted_iota(jnp.int32, (BN, 1), 0) + i * BN
        msk = (rows < N).astype(jnp.float32)
        agg = (a0_ref[...] + a1_ref[...]) * b_ref[...]
        h = jnp.maximum(agg + q_ref[...], 0.0) * msk
        gid = lax.broadcasted_iota(jnp.int32, (BN, G), 1)
        onehot = (seg_ref[...] == gid).astype(jnp.float32)
        sums[...] += lax.dot_general(onehot, h, (((0,), (0,)), ((), ())),
                                     preferred_element_type=jnp.float32)
        counts[...] += lax.dot_general(onehot, jnp.ones((BN, 16), jnp.float32),
                                       (((0,), (0,)), ((), ())),
                                       preferred_element_type=jnp.float32)

        @pl.when(i == NBLK - 1)
        def _():
            pooled = sums[...] / jnp.maximum(counts[...], 1.0)
            out_ref[...] = jnp.dot(pooled, wo_ref[...],
                                   preferred_element_type=jnp.float32) + bo_ref[...]

    return pl.pallas_call(
        body,
        grid=(NBLK,),
        in_specs=[
            pl.BlockSpec((BN, 16), lambda i: (i, 0)),
            pl.BlockSpec((BN, 16), lambda i: (i, 0)),
            pl.BlockSpec((BN, 1), lambda i: (i, 0)),
            pl.BlockSpec((BN, 16), lambda i: (i, 0)),
            pl.BlockSpec((BN, 1), lambda i: (i, 0)),
            pl.BlockSpec((16, 1), lambda i: (0, 0)),
            pl.BlockSpec((1, 1), lambda i: (0, 0)),
        ],
        out_specs=pl.BlockSpec((G, 1), lambda i: (0, 0)),
        out_shape=jax.ShapeDtypeStruct((G, 1), jnp.float32),
        scratch_shapes=[
            pltpu.VMEM((G, 16), jnp.float32),
            pltpu.VMEM((G, 16), jnp.float32),
        ],
    )(acc0, acc1, bvec, q3, seg2, Wo, bo)


def kernel(x, edge_index, segment_ids, W1_0, W2_0, b_0, W1_1, W2_1, b_1,
           W1_2, W2_2, b_2, W1_3, W2_3, b_3, Wo, bo):
    f32 = jnp.float32
    i32 = jnp.int32
    pad_e = jnp.full((2, E_PAD - E), N, i32)
    edges3 = jnp.concatenate([edge_index.astype(i32), pad_e], axis=1) \
        .reshape(2, E_PAD // CHUNK, CHUNK).transpose(1, 0, 2)
    x_pad = jnp.pad(x, ((0, N_PAD - N), (0, 0)))
    seg2 = jnp.concatenate(
        [segment_ids.astype(i32), jnp.full((N_PAD - N,), G, i32)]).reshape(N_PAD, 1)
    ones_pat = (jnp.zeros((2, CHUNK, 16), f32)
                .at[0, :, 0].set(1.0).at[1, :, 1].set(1.0))

    deg = _sc_degrees(edges3, ones_pat, jnp.zeros((N_PAD, 16), f32))
    w1x, q = _tc_pre(x_pad, W1_0, W2_0, b_0.reshape(1, 64))
    a, b, p = _tc_prep(deg[0], deg[1], w1x)

    layers = [
        (64, 32, W1_1, W2_1, b_1),
        (32, 16, W1_2, W2_2, b_2),
        (16, 16, W1_3, W2_3, b_3),
    ]
    for (dout, dnext, W1n, W2n, biasn) in layers:
        acc = _sc_scatter(p, edges3, jnp.zeros((N_PAD, dout), f32), dout)
        p, q = _tc_post(acc[0], acc[1], b, q, a, W1n, W2n,
                        biasn.reshape(1, dnext), dout, dnext)
    acc = _sc_scatter(p, edges3, jnp.zeros((N_PAD, 16), f32), 16)
    return _tc_final(acc[0], acc[1], b, q, seg2, Wo, bo.reshape(1, 1))


# speedup vs baseline: 1.4483x; 1.2084x over previous
import functools

import jax
import jax.numpy as jnp
from jax import lax
from jax.experimental import pallas as pl
from jax.experimental.pallas import tpu as pltpu
from jax.experimental.pallas import tpu_sc as plsc

N = 10000
E = 160000
G = 64
N_PAD = 10240
E_PAD = 163840
CHUNK = 128
NW = 32
NSUB = 16
EP_W = E_PAD // NW
NCHUNK = EP_W // CHUNK
ROWS_W = N_PAD // NSUB
BN = 2048
NBLK = N_PAD // BN


def _sc_mesh():
    return plsc.VectorSubcoreMesh(core_axis_name="c", subcore_axis_name="s",
                                  num_cores=2, num_subcores=NSUB)


K_PIPE = 10


def _sc_degrees(edges3, ones_pat, zeros16):
    @functools.partial(
        pl.kernel,
        out_type=jax.ShapeDtypeStruct((2, N_PAD, 16), jnp.float32),
        mesh=_sc_mesh(),
        scratch_types=[
            pltpu.VMEM((K_PIPE, 2, CHUNK), jnp.int32),
            pltpu.VMEM((CHUNK, 16), jnp.float32),
            pltpu.VMEM((CHUNK, 16), jnp.float32),
            pltpu.SemaphoreType.DMA,
            pltpu.SemaphoreType.DMA,
            pltpu.VMEM_SHARED((N_PAD, 16), jnp.float32),
        ],
    )
    def k(ed_ref, ones_ref, zero_ref, out_ref, idx, e_src, e_dst, sem_i,
          sem_sc, acc):
        cid = lax.axis_index("c")
        sid = lax.axis_index("s")
        cbase = (cid * NSUB + sid) * NCHUNK
        r0 = sid * ROWS_W
        pltpu.sync_copy(ones_ref.at[0], e_src)
        pltpu.sync_copy(ones_ref.at[1], e_dst)
        pltpu.sync_copy(zero_ref.at[pl.ds(r0, ROWS_W)], acc.at[pl.ds(r0, ROWS_W)])
        plsc.subcore_barrier()

        def body(j, carry):
            c0 = cbase + j * K_PIPE
            pltpu.async_copy(ed_ref.at[pl.ds(c0, K_PIPE)], idx, sem_i).wait()
            scats = []
            for s in range(K_PIPE):
                scats.append(pltpu.async_copy(
                    e_src, acc.at[idx.at[s, 0]], sem_sc, add=True))
                scats.append(pltpu.async_copy(
                    e_dst, acc.at[idx.at[s, 1]], sem_sc, add=True))
            for d in scats:
                d.wait()
            return carry

        lax.fori_loop(0, NCHUNK // K_PIPE, body, 0)
        plsc.subcore_barrier()
        pltpu.sync_copy(acc.at[pl.ds(r0, ROWS_W)], out_ref.at[cid, pl.ds(r0, ROWS_W)])

    return k(edges3, ones_pat, zeros16)


def _sc_scatter(p_hbm, edges3, zeros, dout):
    kp = 2 * K_PIPE
    halves = 2 if dout >= 64 else 1
    w = dout // halves

    @functools.partial(
        pl.kernel,
        out_type=jax.ShapeDtypeStruct((2, N_PAD, dout), jnp.float32),
        mesh=_sc_mesh(),
        compiler_params=pltpu.CompilerParams(use_tc_tiling_on_sc=False),
        scratch_types=[
            pltpu.VMEM((kp, 2, CHUNK), jnp.int32),
            [pltpu.VMEM((CHUNK, w), jnp.float32)] * kp,
            pltpu.SemaphoreType.DMA,
            [pltpu.SemaphoreType.DMA] * kp,
            pltpu.SemaphoreType.DMA,
            pltpu.VMEM_SHARED((N_PAD, w), jnp.float32),
            pltpu.VMEM_SHARED((N_PAD, w), jnp.float32),
        ],
    )
    def k(p_ref, ed_ref, zero_ref, out_ref, idx, rows, sem_i, sem_g, sem_sc,
          acc, p_sh):
        cid = lax.axis_index("c")
        sid = lax.axis_index("s")
        cbase = (cid * NSUB + sid) * NCHUNK
        r0 = sid * ROWS_W

        for half in range(halves):
            c_off = half * w
            if halves == 1:
                pltpu.sync_copy(p_ref.at[pl.ds(r0, ROWS_W)],
                                p_sh.at[pl.ds(r0, ROWS_W)])
            else:
                pltpu.sync_copy(p_ref.at[pl.ds(r0, ROWS_W), pl.ds(c_off, w)],
                                p_sh.at[pl.ds(r0, ROWS_W)])
            pltpu.sync_copy(zero_ref.at[pl.ds(r0, ROWS_W)],
                            acc.at[pl.ds(r0, ROWS_W)])
            plsc.subcore_barrier()

            def body(j, carry):
                c0 = cbase + j * kp
                pltpu.async_copy(ed_ref.at[pl.ds(c0, kp)], idx, sem_i).wait()
                gathers = [
                    pltpu.async_copy(p_sh.at[idx.at[s, 0]], rows[s], sem_g[s])
                    for s in range(kp)
                ]
                scats = []
                for s in range(kp):
                    gathers[s].wait()
                    scats.append(pltpu.async_copy(
                        rows[s], acc.at[idx.at[s, 1]], sem_sc, add=True))
                for d in scats:
                    d.wait()
                return carry

            lax.fori_loop(0, NCHUNK // kp, body, 0)
            plsc.subcore_barrier()
            if halves == 1:
                pltpu.sync_copy(acc.at[pl.ds(r0, ROWS_W)],
                                out_ref.at[cid, pl.ds(r0, ROWS_W)])
            else:
                pltpu.sync_copy(acc.at[pl.ds(r0, ROWS_W)],
                                out_ref.at[cid, pl.ds(r0, ROWS_W),
                                           pl.ds(c_off, w)])

    return k(p_hbm, edges3, zeros)


def _tc_pre(x_pad, W1_0, W2_0, b_0):
    def body(x_ref, w1_ref, w2_ref, bias_ref, w1x_ref, q_ref):
        xb = x_ref[...]
        w1x_ref[...] = jnp.dot(xb, w1_ref[...],
                               preferred_element_type=jnp.float32)
        q_ref[...] = jnp.dot(xb, w2_ref[...],
                             preferred_element_type=jnp.float32) + bias_ref[...]

    return pl.pallas_call(
        body,
        grid=(NBLK,),
        in_specs=[
            pl.BlockSpec((BN, 256), lambda i: (i, 0)),
            pl.BlockSpec((256, 64), lambda i: (0, 0)),
            pl.BlockSpec((256, 64), lambda i: (0, 0)),
            pl.BlockSpec((1, 64), lambda i: (0, 0)),
        ],
        out_specs=[
            pl.BlockSpec((BN, 64), lambda i: (i, 0)),
            pl.BlockSpec((BN, 64), lambda i: (i, 0)),
        ],
        out_shape=[
            jax.ShapeDtypeStruct((N_PAD, 64), jnp.float32),
            jax.ShapeDtypeStruct((N_PAD, 64), jnp.float32),
        ],
    )(x_pad, W1_0, W2_0, b_0)


def _tc_prep(d0, d1, w1x):
    def body(d0_ref, d1_ref, w1x_ref, a_ref, b_ref, p_ref):
        deg = d0_ref[...] + d1_ref[...]
        a = lax.rsqrt(jnp.maximum(deg[:, 0:1], 1.0))
        b = lax.rsqrt(jnp.maximum(deg[:, 1:2], 1.0))
        a_ref[...] = a
        b_ref[...] = b
        p_ref[...] = w1x_ref[...] * a

    return pl.pallas_call(
        body,
        grid=(NBLK,),
        in_specs=[
            pl.BlockSpec((BN, 16), lambda i: (i, 0)),
            pl.BlockSpec((BN, 16), lambda i: (i, 0)),
            pl.BlockSpec((BN, 64), lambda i: (i, 0)),
        ],
        out_specs=[
            pl.BlockSpec((BN, 1), lambda i: (i, 0)),
            pl.BlockSpec((BN, 1), lambda i: (i, 0)),
            pl.BlockSpec((BN, 64), lambda i: (i, 0)),
        ],
        out_shape=[
            jax.ShapeDtypeStruct((N_PAD, 1), jnp.float32),
            jax.ShapeDtypeStruct((N_PAD, 1), jnp.float32),
            jax.ShapeDtypeStruct((N_PAD, 64), jnp.float32),
        ],
    )(d0, d1, w1x)


def _tc_post(acc0, acc1, bvec, q, avec, W1n, W2n, biasn, dout, dnext):
    def body(a0_ref, a1_ref, b_ref, q_ref, a_ref, w1n_ref, w2n_ref, bias_ref,
             p_ref, qn_ref):
        i = pl.program_id(0)
        rows = lax.broadcasted_iota(jnp.int32, (BN, 1), 0) + i * BN
        msk = (rows < N).astype(jnp.float32)
        agg = (a0_ref[...] + a1_ref[...]) * b_ref[...]
        h = jnp.maximum(agg + q_ref[...], 0.0) * msk
        p_ref[...] = jnp.dot(h, w1n_ref[...],
                             preferred_element_type=jnp.float32) * a_ref[...]
        qn_ref[...] = jnp.dot(h, w2n_ref[...],
                              preferred_element_type=jnp.float32) + bias_ref[...]

    return pl.pallas_call(
        body,
        grid=(NBLK,),
        in_specs=[
            pl.BlockSpec((BN, dout), lambda i: (i, 0)),
            pl.BlockSpec((BN, dout), lambda i: (i, 0)),
            pl.BlockSpec((BN, 1), lambda i: (i, 0)),
            pl.BlockSpec((BN, dout), lambda i: (i, 0)),
            pl.BlockSpec((BN, 1), lambda i: (i, 0)),
            pl.BlockSpec((dout, dnext), lambda i: (0, 0)),
            pl.BlockSpec((dout, dnext), lambda i: (0, 0)),
            pl.BlockSpec((1, dnext), lambda i: (0, 0)),
        ],
        out_specs=[
            pl.BlockSpec((BN, dnext), lambda i: (i, 0)),
            pl.BlockSpec((BN, dnext), lambda i: (i, 0)),
        ],
        out_shape=[
            jax.ShapeDtypeStruct((N_PAD, dnext), jnp.float32),
            jax.ShapeDtypeStruct((N_PAD, dnext), jnp.float32),
        ],
    )(acc0, acc1, bvec, q, avec, W1n, W2n, biasn)


def _tc_final(acc0, acc1, bvec, q3, seg2, Wo, bo):
    def body(a0_ref, a1_ref, b_ref, q_ref, seg_ref, wo_ref,
             bo_ref, out_ref, sums, counts):
        i = pl.program_id(0)

        @pl.when(i == 0)
        def _():
            sums[...] = jnp.zeros_like(sums)
            counts[...] = jnp.zeros_like(counts)

        rows = lax.broadcasted_iota(jnp.int32, (BN, 1), 0) + i * BN
        msk = (rows < N).astype(jnp.float32)
        agg = (a0_ref[...] + a1_ref[...]) * b_ref[...]
        h = jnp.maximum(agg + q_ref[...], 0.0) * msk
        gid = lax.broadcasted_iota(jnp.int32, (BN, G), 1)
        onehot = (seg_ref[...] == gid).astype(jnp.float32)
        sums[...] += lax.dot_general(onehot, h, (((0,), (0,)), ((), ())),
                                     preferred_element_type=jnp.float32)
        counts[...] += lax.dot_general(onehot, jnp.ones((BN, 16), jnp.float32),
                                       (((0,), (0,)), ((), ())),
                                       preferred_element_type=jnp.float32)

        @pl.when(i == NBLK - 1)
        def _():
            pooled = sums[...] / jnp.maximum(counts[...], 1.0)
            out_ref[...] = jnp.dot(pooled, wo_ref[...],
                                   preferred_element_type=jnp.float32) + bo_ref[...]

    return pl.pallas_call(
        body,
        grid=(NBLK,),
        in_specs=[
            pl.BlockSpec((BN, 16), lambda i: (i, 0)),
            pl.BlockSpec((BN, 16), lambda i: (i, 0)),
            pl.BlockSpec((BN, 1), lambda i: (i, 0)),
            pl.BlockSpec((BN, 16), lambda i: (i, 0)),
            pl.BlockSpec((BN, 1), lambda i: (i, 0)),
            pl.BlockSpec((16, 1), lambda i: (0, 0)),
            pl.BlockSpec((1, 1), lambda i: (0, 0)),
        ],
        out_specs=pl.BlockSpec((G, 1), lambda i: (0, 0)),
        out_shape=jax.ShapeDtypeStruct((G, 1), jnp.float32),
        scratch_shapes=[
            pltpu.VMEM((G, 16), jnp.float32),
            pltpu.VMEM((G, 16), jnp.float32),
        ],
    )(acc0, acc1, bvec, q3, seg2, Wo, bo)


def kernel(x, edge_index, segment_ids, W1_0, W2_0, b_0, W1_1, W2_1, b_1,
           W1_2, W2_2, b_2, W1_3, W2_3, b_3, Wo, bo):
    f32 = jnp.float32
    i32 = jnp.int32
    pad_e = jnp.full((2, E_PAD - E), N, i32)
    edges3 = jnp.concatenate([edge_index.astype(i32), pad_e], axis=1) \
        .reshape(2, E_PAD // CHUNK, CHUNK).transpose(1, 0, 2)
    x_pad = jnp.pad(x, ((0, N_PAD - N), (0, 0)))
    seg2 = jnp.concatenate(
        [segment_ids.astype(i32), jnp.full((N_PAD - N,), G, i32)]).reshape(N_PAD, 1)
    ones_pat = (jnp.zeros((2, CHUNK, 16), f32)
                .at[0, :, 0].set(1.0).at[1, :, 1].set(1.0))

    deg = _sc_degrees(edges3, ones_pat, jnp.zeros((N_PAD, 16), f32))
    w1x, q = _tc_pre(x_pad, W1_0, W2_0, b_0.reshape(1, 64))
    a, b, p = _tc_prep(deg[0], deg[1], w1x)

    layers = [
        (64, 32, W1_1, W2_1, b_1),
        (32, 16, W1_2, W2_2, b_2),
        (16, 16, W1_3, W2_3, b_3),
    ]
    for (dout, dnext, W1n, W2n, biasn) in layers:
        acc = _sc_scatter(p, edges3, jnp.zeros((N_PAD, min(dout, 32)), f32),
                          dout)
        p, q = _tc_post(acc[0], acc[1], b, q, a, W1n, W2n,
                        biasn.reshape(1, dnext), dout, dnext)
    acc = _sc_scatter(p, edges3, jnp.zeros((N_PAD, 16), f32), 16)
    return _tc_final(acc[0], acc[1], b, q, seg2, Wo, bo.reshape(1, 1))


# speedup vs baseline: 1.4488x; 1.0004x over previous
import functools

import jax
import jax.numpy as jnp
from jax import lax
from jax.experimental import pallas as pl
from jax.experimental.pallas import tpu as pltpu
from jax.experimental.pallas import tpu_sc as plsc

N = 10000
E = 160000
G = 64
N_PAD = 10240
E_PAD = 163840
CHUNK = 128
NW = 32
NSUB = 16
EP_W = E_PAD // NW
NCHUNK = EP_W // CHUNK
ROWS_W = N_PAD // NSUB
BN = 2048
NBLK = N_PAD // BN


def _sc_mesh():
    return plsc.VectorSubcoreMesh(core_axis_name="c", subcore_axis_name="s",
                                  num_cores=2, num_subcores=NSUB)


K_PIPE = 10


def _sc_degrees(edges3, ones_pat, zeros16):
    @functools.partial(
        pl.kernel,
        out_type=jax.ShapeDtypeStruct((2, N_PAD, 16), jnp.float32),
        mesh=_sc_mesh(),
        scratch_types=[
            pltpu.VMEM((K_PIPE, 2, CHUNK), jnp.int32),
            pltpu.VMEM((CHUNK, 16), jnp.float32),
            pltpu.VMEM((CHUNK, 16), jnp.float32),
            pltpu.SemaphoreType.DMA,
            pltpu.SemaphoreType.DMA,
            pltpu.VMEM_SHARED((N_PAD, 16), jnp.float32),
        ],
    )
    def k(ed_ref, ones_ref, zero_ref, out_ref, idx, e_src, e_dst,
          sem_i, sem_sc, acc):
        cid = lax.axis_index("c")
        sid = lax.axis_index("s")
        cbase = (cid * NSUB + sid) * NCHUNK
        r0 = sid * ROWS_W
        pltpu.sync_copy(ones_ref.at[0], e_src)
        pltpu.sync_copy(ones_ref.at[1], e_dst)
        pltpu.sync_copy(zero_ref.at[pl.ds(r0, ROWS_W)], acc.at[pl.ds(r0, ROWS_W)])
        plsc.subcore_barrier()

        def body(j, carry):
            c0 = cbase + j * K_PIPE
            pltpu.async_copy(ed_ref.at[pl.ds(c0, K_PIPE)], idx, sem_i).wait()
            scats = []
            for s in range(K_PIPE):
                scats.append(pltpu.async_copy(
                    e_src, acc.at[idx.at[s, 0]], sem_sc, add=True))
                scats.append(pltpu.async_copy(
                    e_dst, acc.at[idx.at[s, 1]], sem_sc, add=True))
            for d in scats:
                d.wait()
            return carry

        lax.fori_loop(0, NCHUNK // K_PIPE, body, 0)
        plsc.subcore_barrier()
        pltpu.sync_copy(acc.at[pl.ds(r0, ROWS_W)], out_ref.at[cid, pl.ds(r0, ROWS_W)])

    return k(edges3, ones_pat, zeros16)


def _sc_scatter(p_hbm, edges3, zeros, dout):
    kp = 2 * K_PIPE
    halves = 2 if dout >= 64 else 1
    w = dout // halves

    @functools.partial(
        pl.kernel,
        out_type=jax.ShapeDtypeStruct((2, N_PAD, dout), jnp.float32),
        mesh=_sc_mesh(),
        compiler_params=pltpu.CompilerParams(use_tc_tiling_on_sc=False),
        scratch_types=[
            pltpu.VMEM((kp, 2, CHUNK), jnp.int32),
            [pltpu.VMEM((CHUNK, w), jnp.float32)] * kp,
            pltpu.SemaphoreType.DMA,
            [pltpu.SemaphoreType.DMA] * kp,
            pltpu.SemaphoreType.DMA,
            pltpu.VMEM_SHARED((N_PAD, w), jnp.float32),
            pltpu.VMEM_SHARED((N_PAD, w), jnp.float32),
        ],
    )
    def k(p_ref, ed_ref, zero_ref, out_ref, idx, rows, sem_i, sem_g,
          sem_sc, acc, p_sh):
        cid = lax.axis_index("c")
        sid = lax.axis_index("s")
        cbase = (cid * NSUB + sid) * NCHUNK
        r0 = sid * ROWS_W

        for half in range(halves):
            c_off = half * w
            if halves == 1:
                pltpu.sync_copy(p_ref.at[pl.ds(r0, ROWS_W)],
                                p_sh.at[pl.ds(r0, ROWS_W)])
            else:
                pltpu.sync_copy(p_ref.at[pl.ds(r0, ROWS_W), pl.ds(c_off, w)],
                                p_sh.at[pl.ds(r0, ROWS_W)])
            pltpu.sync_copy(zero_ref.at[pl.ds(r0, ROWS_W)],
                            acc.at[pl.ds(r0, ROWS_W)])
            plsc.subcore_barrier()

            def body(j, carry):
                c0 = cbase + j * kp
                pltpu.async_copy(ed_ref.at[pl.ds(c0, kp)], idx, sem_i).wait()
                gathers = [
                    pltpu.async_copy(p_sh.at[idx.at[s, 0]], rows[s], sem_g[s])
                    for s in range(kp)
                ]
                scats = []
                for s in range(kp):
                    gathers[s].wait()
                    scats.append(pltpu.async_copy(
                        rows[s], acc.at[idx.at[s, 1]], sem_sc, add=True))
                for d in scats:
                    d.wait()
                return carry

            lax.fori_loop(0, NCHUNK // kp, body, 0)
            plsc.subcore_barrier()
            if halves == 1:
                pltpu.sync_copy(acc.at[pl.ds(r0, ROWS_W)],
                                out_ref.at[cid, pl.ds(r0, ROWS_W)])
            else:
                pltpu.sync_copy(acc.at[pl.ds(r0, ROWS_W)],
                                out_ref.at[cid, pl.ds(r0, ROWS_W),
                                           pl.ds(c_off, w)])

    return k(p_hbm, edges3, zeros)


def _tc_pre(x, W1_0, W2_0, b_0):
    def body(x_ref, w1_ref, w2_ref, bias_ref, w1x_ref, q_ref):
        xb = x_ref[...]
        w1x_ref[...] = jnp.dot(xb, w1_ref[...],
                               preferred_element_type=jnp.float32)
        q_ref[...] = jnp.dot(xb, w2_ref[...],
                             preferred_element_type=jnp.float32) + bias_ref[...]

    return pl.pallas_call(
        body,
        grid=(NBLK,),
        in_specs=[
            pl.BlockSpec((BN, 256), lambda i: (i, 0)),
            pl.BlockSpec((256, 64), lambda i: (0, 0)),
            pl.BlockSpec((256, 64), lambda i: (0, 0)),
            pl.BlockSpec((1, 64), lambda i: (0, 0)),
        ],
        out_specs=[
            pl.BlockSpec((BN, 64), lambda i: (i, 0)),
            pl.BlockSpec((BN, 64), lambda i: (i, 0)),
        ],
        out_shape=[
            jax.ShapeDtypeStruct((N_PAD, 64), jnp.float32),
            jax.ShapeDtypeStruct((N_PAD, 64), jnp.float32),
        ],
    )(x, W1_0, W2_0, b_0)


def _tc_prep(d0, d1, w1x):
    def body(d0_ref, d1_ref, w1x_ref, a_ref, b_ref, p_ref):
        deg = d0_ref[...] + d1_ref[...]
        a = lax.rsqrt(jnp.maximum(deg[:, 0:1], 1.0))
        b = lax.rsqrt(jnp.maximum(deg[:, 1:2], 1.0))
        a_ref[...] = a
        b_ref[...] = b
        p_ref[...] = w1x_ref[...] * a

    return pl.pallas_call(
        body,
        grid=(NBLK,),
        in_specs=[
            pl.BlockSpec((BN, 16), lambda i: (i, 0)),
            pl.BlockSpec((BN, 16), lambda i: (i, 0)),
            pl.BlockSpec((BN, 64), lambda i: (i, 0)),
        ],
        out_specs=[
            pl.BlockSpec((BN, 1), lambda i: (i, 0)),
            pl.BlockSpec((BN, 1), lambda i: (i, 0)),
            pl.BlockSpec((BN, 64), lambda i: (i, 0)),
        ],
        out_shape=[
            jax.ShapeDtypeStruct((N_PAD, 1), jnp.float32),
            jax.ShapeDtypeStruct((N_PAD, 1), jnp.float32),
            jax.ShapeDtypeStruct((N_PAD, 64), jnp.float32),
        ],
    )(d0, d1, w1x)


def _tc_post(acc0, acc1, bvec, q, avec, W1n, W2n, biasn, dout, dnext):
    def body(a0_ref, a1_ref, b_ref, q_ref, a_ref, w1n_ref, w2n_ref, bias_ref,
             p_ref, qn_ref):
        i = pl.program_id(0)
        rows = lax.broadcasted_iota(jnp.int32, (BN, 1), 0) + i * BN
        agg = (a0_ref[...] + a1_ref[...]) * b_ref[...]
        h = jnp.where(rows < N, jnp.maximum(agg + q_ref[...], 0.0), 0.0)
        p_ref[...] = jnp.dot(h, w1n_ref[...],
                             preferred_element_type=jnp.float32) * a_ref[...]
        qn_ref[...] = jnp.dot(h, w2n_ref[...],
                              preferred_element_type=jnp.float32) + bias_ref[...]

    return pl.pallas_call(
        body,
        grid=(NBLK,),
        in_specs=[
            pl.BlockSpec((BN, dout), lambda i: (i, 0)),
            pl.BlockSpec((BN, dout), lambda i: (i, 0)),
            pl.BlockSpec((BN, 1), lambda i: (i, 0)),
            pl.BlockSpec((BN, dout), lambda i: (i, 0)),
            pl.BlockSpec((BN, 1), lambda i: (i, 0)),
            pl.BlockSpec((dout, dnext), lambda i: (0, 0)),
            pl.BlockSpec((dout, dnext), lambda i: (0, 0)),
            pl.BlockSpec((1, dnext), lambda i: (0, 0)),
        ],
        out_specs=[
            pl.BlockSpec((BN, dnext), lambda i: (i, 0)),
            pl.BlockSpec((BN, dnext), lambda i: (i, 0)),
        ],
        out_shape=[
            jax.ShapeDtypeStruct((N_PAD, dnext), jnp.float32),
            jax.ShapeDtypeStruct((N_PAD, dnext), jnp.float32),
        ],
    )(acc0, acc1, bvec, q, avec, W1n, W2n, biasn)


def _tc_final(acc0, acc1, bvec, q3, seg2, Wo, bo):
    def body(a0_ref, a1_ref, b_ref, q_ref, seg_ref, wo_ref,
             bo_ref, out_ref, sums, counts):
        i = pl.program_id(0)

        @pl.when(i == 0)
        def _():
            sums[...] = jnp.zeros_like(sums)
            counts[...] = jnp.zeros_like(counts)

        rows = lax.broadcasted_iota(jnp.int32, (BN, 1), 0) + i * BN
        agg = (a0_ref[...] + a1_ref[...]) * b_ref[...]
        h = jnp.where(rows < N, jnp.maximum(agg + q_ref[...], 0.0), 0.0)
        gid = lax.broadcasted_iota(jnp.int32, (BN, G), 1)
        onehot = (seg_ref[...] == gid).astype(jnp.float32)
        sums[...] += lax.dot_general(onehot, h, (((0,), (0,)), ((), ())),
                                     preferred_element_type=jnp.float32)
        counts[...] += lax.dot_general(onehot, jnp.ones((BN, 16), jnp.float32),
                                       (((0,), (0,)), ((), ())),
                                       preferred_element_type=jnp.float32)

        @pl.when(i == NBLK - 1)
        def _():
            pooled = sums[...] / jnp.maximum(counts[...], 1.0)
            out_ref[...] = jnp.dot(pooled, wo_ref[...],
                                   preferred_element_type=jnp.float32) + bo_ref[...]

    return pl.pallas_call(
        body,
        grid=(NBLK,),
        in_specs=[
            pl.BlockSpec((BN, 16), lambda i: (i, 0)),
            pl.BlockSpec((BN, 16), lambda i: (i, 0)),
            pl.BlockSpec((BN, 1), lambda i: (i, 0)),
            pl.BlockSpec((BN, 16), lambda i: (i, 0)),
            pl.BlockSpec((BN, 1), lambda i: (i, 0)),
            pl.BlockSpec((16, 1), lambda i: (0, 0)),
            pl.BlockSpec((1, 1), lambda i: (0, 0)),
        ],
        out_specs=pl.BlockSpec((G, 1), lambda i: (0, 0)),
        out_shape=jax.ShapeDtypeStruct((G, 1), jnp.float32),
        scratch_shapes=[
            pltpu.VMEM((G, 16), jnp.float32),
            pltpu.VMEM((G, 16), jnp.float32),
        ],
    )(acc0, acc1, bvec, q3, seg2, Wo, bo)


def kernel(x, edge_index, segment_ids, W1_0, W2_0, b_0, W1_1, W2_1, b_1,
           W1_2, W2_2, b_2, W1_3, W2_3, b_3, Wo, bo):
    f32 = jnp.float32
    i32 = jnp.int32
    pad_e = jnp.full((2, E_PAD - E), N, i32)
    edges3 = jnp.concatenate([edge_index.astype(i32), pad_e], axis=1) \
        .reshape(2, E_PAD // CHUNK, CHUNK).transpose(1, 0, 2)
    seg2 = jnp.concatenate(
        [segment_ids.astype(i32), jnp.full((N_PAD - N,), G, i32)]).reshape(N_PAD, 1)
    ones_pat = (jnp.zeros((2, CHUNK, 16), f32)
                .at[0, :, 0].set(1.0).at[1, :, 1].set(1.0))

    deg = _sc_degrees(edges3, ones_pat, jnp.zeros((N_PAD, 16), f32))
    w1x, q = _tc_pre(x, W1_0, W2_0, b_0.reshape(1, 64))
    a, b, p = _tc_prep(deg[0], deg[1], w1x)

    layers = [
        (64, 32, W1_1, W2_1, b_1),
        (32, 16, W1_2, W2_2, b_2),
        (16, 16, W1_3, W2_3, b_3),
    ]
    for (dout, dnext, W1n, W2n, biasn) in layers:
        acc = _sc_scatter(p, edges3, jnp.zeros((N_PAD, min(dout, 32)), f32),
                          dout)
        p, q = _tc_post(acc[0], acc[1], b, q, a, W1n, W2n,
                        biasn.reshape(1, dnext), dout, dnext)
    acc = _sc_scatter(p, edges3, jnp.zeros((N_PAD, 16), f32), 16)
    return _tc_final(acc[0], acc[1], b, q, seg2, Wo, bo.reshape(1, 1))


# speedup vs baseline: 1.4827x; 1.0234x over previous
import functools

import jax
import jax.numpy as jnp
from jax import lax
from jax.experimental import pallas as pl
from jax.experimental.pallas import tpu as pltpu
from jax.experimental.pallas import tpu_sc as plsc

N = 10000
E = 160000
G = 64
N_PAD = 10240
E_PAD = 163840
CHUNK = 128
NW = 32
NSUB = 16
EP_W = E_PAD // NW
NCHUNK = EP_W // CHUNK
ROWS_W = N_PAD // NSUB
BN = 5120
NBLK = N_PAD // BN


def _sc_mesh():
    return plsc.VectorSubcoreMesh(core_axis_name="c", subcore_axis_name="s",
                                  num_cores=2, num_subcores=NSUB)


K_PIPE = 10


def _sc_degrees(edges3, ones_pat, zeros16):
    @functools.partial(
        pl.kernel,
        out_type=jax.ShapeDtypeStruct((2, N_PAD, 16), jnp.float32),
        mesh=_sc_mesh(),
        scratch_types=[
            pltpu.VMEM((K_PIPE, 2, CHUNK), jnp.int32),
            pltpu.VMEM((CHUNK, 16), jnp.float32),
            pltpu.VMEM((CHUNK, 16), jnp.float32),
            pltpu.SemaphoreType.DMA,
            pltpu.SemaphoreType.DMA,
            pltpu.VMEM_SHARED((N_PAD, 16), jnp.float32),
        ],
    )
    def k(ed_ref, ones_ref, zero_ref, out_ref, idx, e_src, e_dst,
          sem_i, sem_sc, acc):
        cid = lax.axis_index("c")
        sid = lax.axis_index("s")
        cbase = (cid * NSUB + sid) * NCHUNK
        r0 = sid * ROWS_W
        pltpu.sync_copy(ones_ref.at[0], e_src)
        pltpu.sync_copy(ones_ref.at[1], e_dst)
        pltpu.sync_copy(zero_ref.at[pl.ds(r0, ROWS_W)], acc.at[pl.ds(r0, ROWS_W)])
        plsc.subcore_barrier()

        def body(j, carry):
            c0 = cbase + j * K_PIPE
            pltpu.async_copy(ed_ref.at[pl.ds(c0, K_PIPE)], idx, sem_i).wait()
            scats = []
            for s in range(K_PIPE):
                scats.append(pltpu.async_copy(
                    e_src, acc.at[idx.at[s, 0]], sem_sc, add=True))
                scats.append(pltpu.async_copy(
                    e_dst, acc.at[idx.at[s, 1]], sem_sc, add=True))
            for d in scats:
                d.wait()
            return carry

        lax.fori_loop(0, NCHUNK // K_PIPE, body, 0)
        plsc.subcore_barrier()
        pltpu.sync_copy(acc.at[pl.ds(r0, ROWS_W)], out_ref.at[cid, pl.ds(r0, ROWS_W)])

    return k(edges3, ones_pat, zeros16)


def _sc_scatter(p_hbm, edges3, zeros, dout):
    kp = 2 * K_PIPE
    halves = 2 if dout >= 64 else 1
    w = dout // halves

    @functools.partial(
        pl.kernel,
        out_type=jax.ShapeDtypeStruct((2, N_PAD, dout), jnp.float32),
        mesh=_sc_mesh(),
        compiler_params=pltpu.CompilerParams(use_tc_tiling_on_sc=False),
        scratch_types=[
            pltpu.VMEM((kp, 2, CHUNK), jnp.int32),
            [pltpu.VMEM((CHUNK, w), jnp.float32)] * kp,
            pltpu.SemaphoreType.DMA,
            [pltpu.SemaphoreType.DMA] * kp,
            pltpu.SemaphoreType.DMA,
            pltpu.VMEM_SHARED((N_PAD, w), jnp.float32),
            pltpu.VMEM_SHARED((N_PAD, w), jnp.float32),
        ],
    )
    def k(p_ref, ed_ref, zero_ref, out_ref, idx, rows, sem_i, sem_g,
          sem_sc, acc, p_sh):
        cid = lax.axis_index("c")
        sid = lax.axis_index("s")
        cbase = (cid * NSUB + sid) * NCHUNK
        r0 = sid * ROWS_W

        for half in range(halves):
            c_off = half * w
            if halves == 1:
                pltpu.sync_copy(p_ref.at[pl.ds(r0, ROWS_W)],
                                p_sh.at[pl.ds(r0, ROWS_W)])
            else:
                pltpu.sync_copy(p_ref.at[pl.ds(r0, ROWS_W), pl.ds(c_off, w)],
                                p_sh.at[pl.ds(r0, ROWS_W)])
            pltpu.sync_copy(zero_ref.at[pl.ds(r0, ROWS_W)],
                            acc.at[pl.ds(r0, ROWS_W)])
            plsc.subcore_barrier()

            def body(j, carry):
                c0 = cbase + j * kp
                pltpu.async_copy(ed_ref.at[pl.ds(c0, kp)], idx, sem_i).wait()
                gathers = [
                    pltpu.async_copy(p_sh.at[idx.at[s, 0]], rows[s], sem_g[s])
                    for s in range(kp)
                ]
                scats = []
                for s in range(kp):
                    gathers[s].wait()
                    scats.append(pltpu.async_copy(
                        rows[s], acc.at[idx.at[s, 1]], sem_sc, add=True))
                for d in scats:
                    d.wait()
                return carry

            lax.fori_loop(0, NCHUNK // kp, body, 0)
            plsc.subcore_barrier()
            if halves == 1:
                pltpu.sync_copy(acc.at[pl.ds(r0, ROWS_W)],
                                out_ref.at[cid, pl.ds(r0, ROWS_W)])
            else:
                pltpu.sync_copy(acc.at[pl.ds(r0, ROWS_W)],
                                out_ref.at[cid, pl.ds(r0, ROWS_W),
                                           pl.ds(c_off, w)])

    return k(p_hbm, edges3, zeros)


def _tc_pre(x, W1_0, W2_0, b_0):
    def body(x_ref, w1_ref, w2_ref, bias_ref, w1x_ref, q_ref):
        xb = x_ref[...]
        w1x_ref[...] = jnp.dot(xb, w1_ref[...],
                               preferred_element_type=jnp.float32)
        q_ref[...] = jnp.dot(xb, w2_ref[...],
                             preferred_element_type=jnp.float32) + bias_ref[...]

    return pl.pallas_call(
        body,
        grid=(NBLK,),
        in_specs=[
            pl.BlockSpec((BN, 256), lambda i: (i, 0)),
            pl.BlockSpec((256, 64), lambda i: (0, 0)),
            pl.BlockSpec((256, 64), lambda i: (0, 0)),
            pl.BlockSpec((1, 64), lambda i: (0, 0)),
        ],
        out_specs=[
            pl.BlockSpec((BN, 64), lambda i: (i, 0)),
            pl.BlockSpec((BN, 64), lambda i: (i, 0)),
        ],
        out_shape=[
            jax.ShapeDtypeStruct((N_PAD, 64), jnp.float32),
            jax.ShapeDtypeStruct((N_PAD, 64), jnp.float32),
        ],
    )(x, W1_0, W2_0, b_0)


def _tc_prep(d0, d1, w1x):
    def body(d0_ref, d1_ref, w1x_ref, a_ref, b_ref, p_ref):
        deg = d0_ref[...] + d1_ref[...]
        a = lax.rsqrt(jnp.maximum(deg[:, 0:1], 1.0))
        b = lax.rsqrt(jnp.maximum(deg[:, 1:2], 1.0))
        a_ref[...] = a
        b_ref[...] = b
        p_ref[...] = w1x_ref[...] * a

    return pl.pallas_call(
        body,
        grid=(NBLK,),
        in_specs=[
            pl.BlockSpec((BN, 16), lambda i: (i, 0)),
            pl.BlockSpec((BN, 16), lambda i: (i, 0)),
            pl.BlockSpec((BN, 64), lambda i: (i, 0)),
        ],
        out_specs=[
            pl.BlockSpec((BN, 1), lambda i: (i, 0)),
            pl.BlockSpec((BN, 1), lambda i: (i, 0)),
            pl.BlockSpec((BN, 64), lambda i: (i, 0)),
        ],
        out_shape=[
            jax.ShapeDtypeStruct((N_PAD, 1), jnp.float32),
            jax.ShapeDtypeStruct((N_PAD, 1), jnp.float32),
            jax.ShapeDtypeStruct((N_PAD, 64), jnp.float32),
        ],
    )(d0, d1, w1x)


def _tc_post(acc0, acc1, bvec, q, avec, W1n, W2n, biasn, dout, dnext):
    def body(a0_ref, a1_ref, b_ref, q_ref, a_ref, w1n_ref, w2n_ref, bias_ref,
             p_ref, qn_ref):
        i = pl.program_id(0)
        rows = lax.broadcasted_iota(jnp.int32, (BN, 1), 0) + i * BN
        agg = (a0_ref[...] + a1_ref[...]) * b_ref[...]
        h = jnp.where(rows < N, jnp.maximum(agg + q_ref[...], 0.0), 0.0)
        p_ref[...] = jnp.dot(h, w1n_ref[...],
                             preferred_element_type=jnp.float32) * a_ref[...]
        qn_ref[...] = jnp.dot(h, w2n_ref[...],
                              preferred_element_type=jnp.float32) + bias_ref[...]

    return pl.pallas_call(
        body,
        grid=(NBLK,),
        in_specs=[
            pl.BlockSpec((BN, dout), lambda i: (i, 0)),
            pl.BlockSpec((BN, dout), lambda i: (i, 0)),
            pl.BlockSpec((BN, 1), lambda i: (i, 0)),
            pl.BlockSpec((BN, dout), lambda i: (i, 0)),
            pl.BlockSpec((BN, 1), lambda i: (i, 0)),
            pl.BlockSpec((dout, dnext), lambda i: (0, 0)),
            pl.BlockSpec((dout, dnext), lambda i: (0, 0)),
            pl.BlockSpec((1, dnext), lambda i: (0, 0)),
        ],
        out_specs=[
            pl.BlockSpec((BN, dnext), lambda i: (i, 0)),
            pl.BlockSpec((BN, dnext), lambda i: (i, 0)),
        ],
        out_shape=[
            jax.ShapeDtypeStruct((N_PAD, dnext), jnp.float32),
            jax.ShapeDtypeStruct((N_PAD, dnext), jnp.float32),
        ],
    )(acc0, acc1, bvec, q, avec, W1n, W2n, biasn)


def _tc_final(acc0, acc1, bvec, q3, seg2, Wo, bo):
    def body(a0_ref, a1_ref, b_ref, q_ref, seg_ref, wo_ref,
             bo_ref, out_ref, sums, counts):
        i = pl.program_id(0)

        @pl.when(i == 0)
        def _():
            sums[...] = jnp.zeros_like(sums)
            counts[...] = jnp.zeros_like(counts)

        rows = lax.broadcasted_iota(jnp.int32, (BN, 1), 0) + i * BN
        agg = (a0_ref[...] + a1_ref[...]) * b_ref[...]
        h = jnp.where(rows < N, jnp.maximum(agg + q_ref[...], 0.0), 0.0)
        gid = lax.broadcasted_iota(jnp.int32, (BN, G), 1)
        onehot = (seg_ref[...] == gid).astype(jnp.float32)
        sums[...] += lax.dot_general(onehot, h, (((0,), (0,)), ((), ())),
                                     preferred_element_type=jnp.float32)
        counts[...] += lax.dot_general(onehot, jnp.ones((BN, 16), jnp.float32),
                                       (((0,), (0,)), ((), ())),
                                       preferred_element_type=jnp.float32)

        @pl.when(i == NBLK - 1)
        def _():
            pooled = sums[...] / jnp.maximum(counts[...], 1.0)
            out_ref[...] = jnp.dot(pooled, wo_ref[...],
                                   preferred_element_type=jnp.float32) + bo_ref[...]

    return pl.pallas_call(
        body,
        grid=(NBLK,),
        in_specs=[
            pl.BlockSpec((BN, 16), lambda i: (i, 0)),
            pl.BlockSpec((BN, 16), lambda i: (i, 0)),
            pl.BlockSpec((BN, 1), lambda i: (i, 0)),
            pl.BlockSpec((BN, 16), lambda i: (i, 0)),
            pl.BlockSpec((BN, 1), lambda i: (i, 0)),
            pl.BlockSpec((16, 1), lambda i: (0, 0)),
            pl.BlockSpec((1, 1), lambda i: (0, 0)),
        ],
        out_specs=pl.BlockSpec((G, 1), lambda i: (0, 0)),
        out_shape=jax.ShapeDtypeStruct((G, 1), jnp.float32),
        scratch_shapes=[
            pltpu.VMEM((G, 16), jnp.float32),
            pltpu.VMEM((G, 16), jnp.float32),
        ],
    )(acc0, acc1, bvec, q3, seg2, Wo, bo)


def kernel(x, edge_index, segment_ids, W1_0, W2_0, b_0, W1_1, W2_1, b_1,
           W1_2, W2_2, b_2, W1_3, W2_3, b_3, Wo, bo):
    f32 = jnp.float32
    i32 = jnp.int32
    pad_e = jnp.full((2, E_PAD - E), N, i32)
    edges3 = jnp.concatenate([edge_index.astype(i32), pad_e], axis=1) \
        .reshape(2, E_PAD // CHUNK, CHUNK).transpose(1, 0, 2)
    seg2 = jnp.concatenate(
        [segment_ids.astype(i32), jnp.full((N_PAD - N,), G, i32)]).reshape(N_PAD, 1)
    ones_pat = (jnp.zeros((2, CHUNK, 16), f32)
                .at[0, :, 0].set(1.0).at[1, :, 1].set(1.0))

    deg = _sc_degrees(edges3, ones_pat, jnp.zeros((N_PAD, 16), f32))
    w1x, q = _tc_pre(x, W1_0, W2_0, b_0.reshape(1, 64))
    a, b, p = _tc_prep(deg[0], deg[1], w1x)

    layers = [
        (64, 32, W1_1, W2_1, b_1),
        (32, 16, W1_2, W2_2, b_2),
        (16, 16, W1_3, W2_3, b_3),
    ]
    for (dout, dnext, W1n, W2n, biasn) in layers:
        acc = _sc_scatter(p, edges3, jnp.zeros((N_PAD, min(dout, 32)), f32),
                          dout)
        p, q = _tc_post(acc[0], acc[1], b, q, a, W1n, W2n,
                        biasn.reshape(1, dnext), dout, dnext)
    acc = _sc_scatter(p, edges3, jnp.zeros((N_PAD, 16), f32), 16)
    return _tc_final(acc[0], acc[1], b, q, seg2, Wo, bo.reshape(1, 1))
